# baseline (speedup 1.0000x reference)
import functools

import jax
import jax.numpy as jnp
from jax import lax
from jax.experimental import pallas as pl
from jax.experimental.pallas import tpu as pltpu
from jax.experimental.pallas import tpu_sc as plsc

NC, NS = 2, 16
NW = NC * NS
CHUNK = 128
BN = 8192


QN = BN // 4


def _tpose_body(tt_ref, out_ref):
    x = tt_ref[...]
    d = x.shape[0]
    acc = None
    for q in range(4):
        xq = x[:, q * QN:(q + 1) * QN]
        eq = jnp.eye(d, 128, k=q * d, dtype=jnp.float32)
        yq = jax.lax.dot_general(
            xq, eq, (((0,), (0,)), ((), ())),
            preferred_element_type=jnp.float32,
        )
        acc = yq if acc is None else acc + yq
    out_ref[...] = acc


def _tpose(tableT):
    d, v = tableT.shape
    nblk = pl.cdiv(v, BN)
    return pl.pallas_call(
        _tpose_body,
        grid=(nblk,),
        in_specs=[pl.BlockSpec((d, BN), lambda i: (0, i))],
        out_specs=pl.BlockSpec((QN, 128), lambda i: (i, 0)),
        out_shape=jax.ShapeDtypeStruct((nblk * QN, 128), jnp.float32),
    )(tableT)


def _sc_gather(idx3, table_lin, total, d):
    nchunk = idx3.shape[1]
    mesh = plsc.VectorSubcoreMesh(
        core_axis_name="c", subcore_axis_name="s", num_cores=NC, num_subcores=NS
    )
    nbuf = 4
    assert nchunk % nbuf == 0

    def body(idx_hbm, table_hbm, out_hbm, idx_v, rows_v, sems):
        wid = lax.axis_index("s") * NC + lax.axis_index("c")
        pltpu.sync_copy(idx_hbm.at[wid], idx_v)
        base = wid * (nchunk * CHUNK)

        def step(j2, carry):
            j0 = j2 * nbuf
            cps = [
                pltpu.async_copy(
                    table_hbm.at[idx_v.at[j0 + b]], rows_v.at[b], sems.at[b]
                )
                for b in range(nbuf)
            ]
            for b in range(nbuf):
                cps[b].wait()
                pltpu.sync_copy(
                    rows_v.at[b],
                    out_hbm.at[pl.ds(base + (j0 + b) * CHUNK, CHUNK)],
                )
            return carry

        lax.fori_loop(0, nchunk // nbuf, step, 0)

    run = pl.kernel(
        body,
        out_type=jax.ShapeDtypeStruct((total, d), jnp.float32),
        mesh=mesh,
        scratch_types=[
            pltpu.VMEM((nchunk, CHUNK), jnp.int32),
            pltpu.VMEM((nbuf, CHUNK, d), jnp.float32),
            pltpu.SemaphoreType.DMA((nbuf,)),
        ],
        compiler_params=pltpu.CompilerParams(use_tc_tiling_on_sc=False),
    )
    return run(idx3, table_lin)


def _mlp_body(emb_ref, w1_ref, b1_ref, w2_ref, b2_ref, out_ref):
    h = jnp.dot(
        emb_ref[...].astype(jnp.bfloat16),
        w1_ref[...].astype(jnp.bfloat16),
        preferred_element_type=jnp.float32,
    )
    h = jnp.maximum(h + b1_ref[...], 0.0)
    o = jnp.dot(h, w2_ref[...], preferred_element_type=jnp.float32) + b2_ref[...]
    out_ref[...] = jax.nn.sigmoid(o)


def _mlp(emb, W1, b1, W2, b2, block_m=1024):
    B, d_in = emb.shape
    d_hid = W1.shape[1]
    d_out = W2.shape[1]
    grid = (B // block_m,)
    return pl.pallas_call(
        _mlp_body,
        grid=grid,
        in_specs=[
            pl.BlockSpec((block_m, d_in), lambda i: (i, 0)),
            pl.BlockSpec((d_in, d_hid), lambda i: (0, 0)),
            pl.BlockSpec((1, d_hid), lambda i: (0, 0)),
            pl.BlockSpec((d_hid, d_out), lambda i: (0, 0)),
            pl.BlockSpec((1, d_out), lambda i: (0, 0)),
        ],
        out_specs=pl.BlockSpec((block_m, d_out), lambda i: (i, 0)),
        out_shape=jax.ShapeDtypeStruct((B, d_out), jnp.float32),
    )(emb, W1, b1.reshape(1, d_hid), W2,
      b2.reshape(1, d_out))


def kernel(input, table, W1, b1, W2, b2):
    B, K = input.shape
    V, d = table.shape
    total = B * K
    per_w = total // NW
    nchunk = per_w // CHUNK
    r = input
    fr = (r & ~(BN - 1)) + 4 * (r & (QN - 1)) + ((r & (BN - 1)) >> 11)
    idx3 = fr.reshape(NW, nchunk, CHUNK)
    tp = _tpose(table.T)
    table_lin = tp.reshape(tp.shape[0] * 4, d)
    rows = _sc_gather(idx3, table_lin, total, d)
    emb = rows.reshape(B, K * d)
    return _mlp(emb, W1, b1, W2, b2)


# speedup vs baseline: 1.0020x; 1.0020x over previous
import functools

import jax
import jax.numpy as jnp
from jax import lax
from jax.experimental import pallas as pl
from jax.experimental.pallas import tpu as pltpu
from jax.experimental.pallas import tpu_sc as plsc

NC, NS = 2, 16
NW = NC * NS
CHUNK = 128
BN = 8192


QN = BN // 4


def _tpose_body(tt_ref, out_ref):
    x = tt_ref[...]
    d = x.shape[0]
    acc = None
    for q in range(4):
        xq = x[:, q * QN:(q + 1) * QN]
        eq = jnp.eye(d, 128, k=q * d, dtype=jnp.float32)
        yq = jax.lax.dot_general(
            xq, eq, (((0,), (0,)), ((), ())),
            preferred_element_type=jnp.float32,
        )
        acc = yq if acc is None else acc + yq
    out_ref[...] = acc


def _tpose(tableT):
    d, v = tableT.shape
    nblk = pl.cdiv(v, BN)
    return pl.pallas_call(
        _tpose_body,
        grid=(nblk,),
        in_specs=[pl.BlockSpec((d, BN), lambda i: (0, i))],
        out_specs=pl.BlockSpec((QN, 128), lambda i: (i, 0)),
        out_shape=jax.ShapeDtypeStruct((nblk * QN, 128), jnp.float32),
    )(tableT)


def _sc_gather(idx3, table_lin, total, d):
    nchunk = idx3.shape[1]
    mesh = plsc.VectorSubcoreMesh(
        core_axis_name="c", subcore_axis_name="s", num_cores=NC, num_subcores=NS
    )
    nbuf = 4
    assert nchunk % nbuf == 0

    def body(idx_hbm, table_hbm, out_hbm, idx_v, rows_v, sems):
        wid = lax.axis_index("s") * NC + lax.axis_index("c")
        pltpu.sync_copy(idx_hbm.at[wid], idx_v)
        base = wid * (nchunk * CHUNK)

        def step(j2, carry):
            j0 = j2 * nbuf
            cps = [
                pltpu.async_copy(
                    table_hbm.at[idx_v.at[j0 + b]], rows_v.at[b], sems.at[b]
                )
                for b in range(nbuf)
            ]
            for b in range(nbuf):
                cps[b].wait()
                pltpu.sync_copy(
                    rows_v.at[b],
                    out_hbm.at[pl.ds(base + (j0 + b) * CHUNK, CHUNK)],
                )
            return carry

        lax.fori_loop(0, nchunk // nbuf, step, 0)

    run = pl.kernel(
        body,
        out_type=jax.ShapeDtypeStruct((total, d), jnp.float32),
        mesh=mesh,
        scratch_types=[
            pltpu.VMEM((nchunk, CHUNK), jnp.int32),
            pltpu.VMEM((nbuf, CHUNK, d), jnp.float32),
            pltpu.SemaphoreType.DMA((nbuf,)),
        ],
        compiler_params=pltpu.CompilerParams(use_tc_tiling_on_sc=False),
    )
    return run(idx3, table_lin)


def _mlp_body(emb_ref, w1_ref, b1_ref, w2_ref, b2_ref, out_ref):
    h = jnp.dot(emb_ref[...], w1_ref[...], preferred_element_type=jnp.float32)
    h = jnp.maximum(h + b1_ref[...], 0.0)
    o = jnp.dot(h, w2_ref[...], preferred_element_type=jnp.float32) + b2_ref[...]
    out_ref[...] = jax.nn.sigmoid(o)


def _mlp(emb, W1, b1, W2, b2, block_m=1024):
    B, d_in = emb.shape
    d_hid = W1.shape[1]
    d_out = W2.shape[1]
    grid = (B // block_m,)
    return pl.pallas_call(
        _mlp_body,
        grid=grid,
        in_specs=[
            pl.BlockSpec((block_m, d_in), lambda i: (i, 0)),
            pl.BlockSpec((d_in, d_hid), lambda i: (0, 0)),
            pl.BlockSpec((1, d_hid), lambda i: (0, 0)),
            pl.BlockSpec((d_hid, d_out), lambda i: (0, 0)),
            pl.BlockSpec((1, d_out), lambda i: (0, 0)),
        ],
        out_specs=pl.BlockSpec((block_m, d_out), lambda i: (i, 0)),
        out_shape=jax.ShapeDtypeStruct((B, d_out), jnp.float32),
    )(emb, W1, b1.reshape(1, d_hid), W2,
      b2.reshape(1, d_out))


def kernel(input, table, W1, b1, W2, b2):
    B, K = input.shape
    V, d = table.shape
    total = B * K
    per_w = total // NW
    nchunk = per_w // CHUNK
    r = input
    fr = (r & ~(BN - 1)) + 4 * (r & (QN - 1)) + ((r & (BN - 1)) >> 11)
    idx3 = fr.reshape(NW, nchunk, CHUNK)
    tp = _tpose(table.T)
    table_lin = tp.reshape(tp.shape[0] * 4, d)
    rows = _sc_gather(idx3, table_lin, total, d)
    emb = rows.reshape(B, K * d)
    return _mlp(emb, W1, b1, W2, b2)
